# initial kernel scaffold (unmeasured)
import jax
import jax.numpy as jnp
from jax import lax
from jax.experimental import pallas as pl
from jax.experimental.pallas import tpu as pltpu


def kernel(x, dest):
    m, n = x.shape
    dest_row = dest.reshape(1, m)
    dest_col = dest.reshape(m, 1)

    def body(x_ref, drow_ref, dcol_ref, out_ref,
             send_buf, recv_buf, send_sem, recv_sem):
        my_x = lax.axis_index("x")
        my_y = lax.axis_index("y")
        peer = (1 - my_x, my_y)

        barrier_sem = pltpu.get_barrier_semaphore()
        pl.semaphore_signal(barrier_sem, inc=1, device_id=peer,
                            device_id_type=pl.DeviceIdType.MESH)
        pl.semaphore_wait(barrier_sem, 1)

        d_row = drow_ref[...]
        d_col = dcol_ref[...]
        keep_row = d_row == my_x
        keep_col = (d_col == my_x).astype(jnp.float32)

        ia = lax.broadcasted_iota(jnp.int32, (m, m), 0)
        ib = lax.broadcasted_iota(jnp.int32, (m, m), 1)
        tri = (ia < ib).astype(jnp.float32)
        kpos = jnp.sum(tri * keep_col, axis=0, keepdims=True).astype(jnp.int32)
        spos = jnp.sum(tri * (1.0 - keep_col), axis=0, keepdims=True).astype(jnp.int32)
        n_keep = jnp.sum(keep_col).astype(jnp.int32)
        n_recv = m - n_keep

        keep_base = jnp.where(my_x == 0, 0, n_recv)
        recv_base = jnp.where(my_x == 0, n_keep, 0)

        x_bf = x_ref[...].astype(jnp.bfloat16)

        p_send = (jnp.logical_not(keep_row) & (spos == ia)).astype(jnp.bfloat16)
        send_buf[...] = jnp.dot(p_send, x_bf,
                                preferred_element_type=jnp.bfloat16)

        rdma = pltpu.make_async_remote_copy(
            src_ref=send_buf,
            dst_ref=recv_buf,
            send_sem=send_sem,
            recv_sem=recv_sem,
            device_id=peer,
            device_id_type=pl.DeviceIdType.MESH,
        )
        rdma.start()

        p_keep = (keep_row & ((kpos + keep_base) == ia)).astype(jnp.bfloat16)
        acc = jnp.dot(p_keep, x_bf, preferred_element_type=jnp.float32)

        rdma.wait()

        shift = (ia == (ib + recv_base)).astype(jnp.bfloat16)
        out_ref[...] = acc + jnp.dot(shift, recv_buf[...],
                                     preferred_element_type=jnp.float32)

    return pl.pallas_call(
        body,
        out_shape=jax.ShapeDtypeStruct((m, n), jnp.float32),
        in_specs=[
            pl.BlockSpec(memory_space=pltpu.VMEM),
            pl.BlockSpec(memory_space=pltpu.VMEM),
            pl.BlockSpec(memory_space=pltpu.VMEM),
        ],
        out_specs=pl.BlockSpec(memory_space=pltpu.VMEM),
        scratch_shapes=[
            pltpu.VMEM((m, n), jnp.bfloat16),
            pltpu.VMEM((m, n), jnp.bfloat16),
            pltpu.SemaphoreType.DMA,
            pltpu.SemaphoreType.DMA,
        ],
        compiler_params=pltpu.CompilerParams(collective_id=0),
    )(x, dest_row, dest_col)


# baseline (device time: 10403 ns/iter reference)
import jax
import jax.numpy as jnp
from jax import lax
from jax.experimental import pallas as pl
from jax.experimental.pallas import tpu as pltpu


def kernel(x, dest):
    m, n = x.shape
    dest_row = dest.reshape(1, m)
    dest_col = dest.reshape(m, 1)

    def body(x_ref, drow_ref, dcol_ref, out_ref,
             send_buf, recv_buf, send_sem, recv_sem):
        my_x = lax.axis_index("x")
        my_y = lax.axis_index("y")
        peer = (1 - my_x, my_y)

        barrier_sem = pltpu.get_barrier_semaphore()
        pl.semaphore_signal(barrier_sem, inc=1, device_id=peer,
                            device_id_type=pl.DeviceIdType.MESH)
        pl.semaphore_wait(barrier_sem, 1)

        d_row = drow_ref[...]
        d_col = dcol_ref[...]
        keep_row = d_row == my_x
        keep_col = (d_col == my_x).astype(jnp.float32)

        ia = lax.broadcasted_iota(jnp.int32, (m, m), 0)
        ib = lax.broadcasted_iota(jnp.int32, (m, m), 1)
        tri = (ia < ib).astype(jnp.float32)
        kpos = jnp.sum(tri * keep_col, axis=0, keepdims=True).astype(jnp.int32)
        spos = jnp.sum(tri * (1.0 - keep_col), axis=0, keepdims=True).astype(jnp.int32)
        n_keep = jnp.sum(keep_col).astype(jnp.int32)
        n_recv = m - n_keep

        keep_base = jnp.where(my_x == 0, 0, n_recv)
        recv_base = jnp.where(my_x == 0, n_keep, 0)

        x_bf = x_ref[...].astype(jnp.bfloat16)

        p_send = (jnp.logical_not(keep_row) & (spos == ia)).astype(jnp.bfloat16)
        send_buf[...] = jnp.dot(
            p_send, x_bf, preferred_element_type=jnp.float32
        ).astype(jnp.bfloat16)

        rdma = pltpu.make_async_remote_copy(
            src_ref=send_buf,
            dst_ref=recv_buf,
            send_sem=send_sem,
            recv_sem=recv_sem,
            device_id=peer,
            device_id_type=pl.DeviceIdType.MESH,
        )
        rdma.start()

        p_keep = (keep_row & ((kpos + keep_base) == ia)).astype(jnp.bfloat16)
        acc = jnp.dot(p_keep, x_bf, preferred_element_type=jnp.float32)

        rdma.wait()

        shift = (ia == (ib + recv_base)).astype(jnp.bfloat16)
        out_ref[...] = acc + jnp.dot(shift, recv_buf[...],
                                     preferred_element_type=jnp.float32)

    return pl.pallas_call(
        body,
        out_shape=jax.ShapeDtypeStruct((m, n), jnp.float32),
        in_specs=[
            pl.BlockSpec(memory_space=pltpu.VMEM),
            pl.BlockSpec(memory_space=pltpu.VMEM),
            pl.BlockSpec(memory_space=pltpu.VMEM),
        ],
        out_specs=pl.BlockSpec(memory_space=pltpu.VMEM),
        scratch_shapes=[
            pltpu.VMEM((m, n), jnp.bfloat16),
            pltpu.VMEM((m, n), jnp.bfloat16),
            pltpu.SemaphoreType.DMA,
            pltpu.SemaphoreType.DMA,
        ],
        compiler_params=pltpu.CompilerParams(collective_id=0),
    )(x, dest_row, dest_col)


# device time: 8485 ns/iter; 1.2260x vs baseline; 1.2260x over previous
import jax
import jax.numpy as jnp
from jax import lax
from jax.experimental import pallas as pl
from jax.experimental.pallas import tpu as pltpu

C = 64


def kernel(x, dest):
    m, n = x.shape
    nch = m // C
    dest_row = dest.reshape(1, m)
    dest_col = dest.reshape(m, 1)

    def body(x_ref, drow_ref, dcol_ref, out_ref,
             stage, send_sems, recv_sems):
        my_x = lax.axis_index("x")
        my_y = lax.axis_index("y")
        peer = (1 - my_x, my_y)

        barrier_sem = pltpu.get_barrier_semaphore()
        pl.semaphore_signal(barrier_sem, inc=1, device_id=peer,
                            device_id_type=pl.DeviceIdType.MESH)
        pl.semaphore_wait(barrier_sem, 1)

        d_row = drow_ref[...]
        d_col = dcol_ref[...]
        keep_row = d_row == my_x
        keep_col = (d_col == my_x).astype(jnp.float32)

        ia = lax.broadcasted_iota(jnp.int32, (m, m), 0)
        ib = lax.broadcasted_iota(jnp.int32, (m, m), 1)
        tri = (ia < ib).astype(jnp.float32)
        kpos = jnp.sum(tri * keep_col, axis=0, keepdims=True).astype(jnp.int32)
        col = lax.broadcasted_iota(jnp.int32, (1, m), 1)
        spos = col - kpos
        n_keep = jnp.sum(keep_col).astype(jnp.int32)
        n_send = m - n_keep
        n_recv = n_send

        keep_base = jnp.where(my_x == 0, 0, n_recv)
        rb_peer = jnp.where(my_x == 0, 0, m - n_send)
        rb_al_peer = (rb_peer // 8) * 8
        pad = rb_peer - rb_al_peer
        padded = pad + n_send
        recv_base = jnp.where(my_x == 0, n_keep, 0)
        rb_al = (recv_base // 8) * 8
        padded_r = recv_base - rb_al + n_recv

        tgt = jnp.where(keep_row, kpos + keep_base, m + pad + spos)
        big_row = lax.broadcasted_iota(jnp.int32, (2 * m, m), 0)
        perm = (big_row == tgt).astype(jnp.bfloat16)
        x_bf = x_ref[...].astype(jnp.bfloat16)
        stage[...] = jnp.dot(
            perm, x_bf, preferred_element_type=jnp.float32
        ).astype(jnp.bfloat16)

        for c in range(nch):
            @pl.when(c * C < padded)
            def _(c=c):
                delta = jnp.maximum(0, rb_al_peer + (c + 1) * C - m)
                src_start = pl.multiple_of(m + c * C - delta, 8)
                dst_start = pl.multiple_of(rb_al_peer + c * C - delta, 8)
                rdma = pltpu.make_async_remote_copy(
                    src_ref=stage.at[pl.ds(src_start, C), :],
                    dst_ref=out_ref.at[pl.ds(dst_start, C), :],
                    send_sem=send_sems.at[c],
                    recv_sem=recv_sems.at[c],
                    device_id=peer,
                    device_id_type=pl.DeviceIdType.MESH,
                )
                rdma.start()

        for c in range(nch):
            @pl.when(c * C < padded_r)
            def _(c=c):
                rdma = pltpu.make_async_remote_copy(
                    src_ref=stage.at[pl.ds(0, C), :],
                    dst_ref=out_ref.at[pl.ds(0, C), :],
                    send_sem=send_sems.at[c],
                    recv_sem=recv_sems.at[c],
                    device_id=peer,
                    device_id_type=pl.DeviceIdType.MESH,
                )
                rdma.wait_recv()

        nch_r = (padded_r + C - 1) // C
        hi = jnp.minimum(rb_al + nch_r * C, m)
        row = lax.broadcasted_iota(jnp.int32, (m, 1), 0)
        cover = (row >= rb_al) & (row < hi)
        out_ref[...] = (
            jnp.where(cover, out_ref[...], jnp.bfloat16(0))
            + stage[pl.ds(0, m), :]
        )

        for c in range(nch):
            @pl.when(c * C < padded)
            def _(c=c):
                rdma = pltpu.make_async_remote_copy(
                    src_ref=stage.at[pl.ds(0, C), :],
                    dst_ref=out_ref.at[pl.ds(0, C), :],
                    send_sem=send_sems.at[c],
                    recv_sem=recv_sems.at[c],
                    device_id=peer,
                    device_id_type=pl.DeviceIdType.MESH,
                )
                rdma.wait_send()

    return pl.pallas_call(
        body,
        out_shape=jax.ShapeDtypeStruct((m, n), jnp.bfloat16),
        in_specs=[
            pl.BlockSpec(memory_space=pltpu.VMEM),
            pl.BlockSpec(memory_space=pltpu.VMEM),
            pl.BlockSpec(memory_space=pltpu.VMEM),
        ],
        out_specs=pl.BlockSpec(memory_space=pltpu.VMEM),
        scratch_shapes=[
            pltpu.VMEM((2 * m, n), jnp.bfloat16),
            pltpu.SemaphoreType.DMA((nch,)),
            pltpu.SemaphoreType.DMA((nch,)),
        ],
        compiler_params=pltpu.CompilerParams(collective_id=0),
    )(x, dest_row, dest_col)
